# baseline (speedup 1.0000x reference)
import functools

import jax
import jax.numpy as jnp
from jax import lax
from jax.experimental import pallas as pl
from jax.experimental.pallas import tpu as pltpu
from jax.experimental.pallas import tpu_sc as plsc

VOCAB = 100000
DIM = 128
BATCH = 1024


def _gather_sc(table, idx):
    info = plsc.get_sparse_core_info()
    nw = info.num_cores * info.num_subcores
    bpw = BATCH // nw
    mesh = plsc.VectorSubcoreMesh(core_axis_name="c", subcore_axis_name="s")

    @functools.partial(
        pl.kernel,
        mesh=mesh,
        out_type=jax.ShapeDtypeStruct((BATCH, DIM), jnp.float32),
        scratch_types=[
            pltpu.VMEM((bpw,), jnp.int32),
            pltpu.VMEM((bpw, DIM), jnp.float32),
            pltpu.SemaphoreType.DMA,
        ],
    )
    def gather_kernel(table_hbm, idx_hbm, out_hbm, idx_v, rows_v, sem):
        wid = lax.axis_index("s") * info.num_cores + lax.axis_index("c")
        base = wid * bpw
        pltpu.sync_copy(idx_hbm.at[pl.ds(base, bpw)], idx_v)
        pltpu.async_copy(table_hbm.at[idx_v], rows_v, sem).wait()
        pltpu.sync_copy(rows_v, out_hbm.at[pl.ds(base, bpw)])

    return gather_kernel(table, idx)


_VT = 2048
_NT = (VOCAB + _VT - 1) // _VT
_NFULL = VOCAB // _VT
_NB = 4


def _proj_main_kernel(e_ref, w_ref, b_ref, o_hbm, *scratch):
    bufs = scratch[:_NB]
    sems = scratch[_NB:]
    j = pl.program_id(0)
    acc = (
        jnp.dot(e_ref[...], w_ref[...], preferred_element_type=jnp.float32)
        + b_ref[...]
    )
    for s in range(_NB):

        @pl.when(jax.lax.rem(j, _NB) == s)
        def _(s=s):
            @pl.when(j >= _NB)
            def _():
                pltpu.make_async_copy(
                    bufs[s], o_hbm.at[:, pl.ds((j - _NB) * _VT, _VT)], sems[s]
                ).wait()

            bufs[s][...] = acc
            pltpu.make_async_copy(
                bufs[s], o_hbm.at[:, pl.ds(j * _VT, _VT)], sems[s]
            ).start()

    @pl.when(j == _NFULL - 1)
    def _():
        for s in range(_NB):
            jl = _NFULL - 1 - ((_NFULL - 1 - s) % _NB)
            pltpu.make_async_copy(
                bufs[s], o_hbm.at[:, pl.ds(jl * _VT, _VT)], sems[s]
            ).wait()


def _proj_tail_kernel(prev_ref, e_ref, w_ref, b_ref, o_ref):
    del prev_ref
    o_ref[...] = (
        jnp.dot(e_ref[...], w_ref[...], preferred_element_type=jnp.float32)
        + b_ref[...]
    )


def _project(e, W, b):
    b2 = b.reshape(1, VOCAB)
    out = pl.pallas_call(
        _proj_main_kernel,
        grid=(_NFULL,),
        in_specs=[
            pl.BlockSpec((BATCH, DIM), lambda j: (0, 0)),
            pl.BlockSpec((DIM, _VT), lambda j: (0, j)),
            pl.BlockSpec((1, _VT), lambda j: (0, j)),
        ],
        out_specs=pl.BlockSpec(memory_space=pl.ANY),
        out_shape=jax.ShapeDtypeStruct((BATCH, VOCAB), jnp.float32),
        scratch_shapes=(
            [pltpu.VMEM((BATCH, _VT), jnp.float32) for _ in range(_NB)]
            + [pltpu.SemaphoreType.DMA for _ in range(_NB)]
        ),
    )(e, W, b2)
    out = pl.pallas_call(
        _proj_tail_kernel,
        grid=(1,),
        in_specs=[
            pl.BlockSpec(memory_space=pl.ANY),
            pl.BlockSpec((BATCH, DIM), lambda j: (0, 0)),
            pl.BlockSpec((DIM, _VT), lambda j: (0, _NFULL)),
            pl.BlockSpec((1, _VT), lambda j: (0, _NFULL)),
        ],
        out_specs=pl.BlockSpec((BATCH, _VT), lambda j: (0, _NFULL)),
        out_shape=jax.ShapeDtypeStruct((BATCH, VOCAB), jnp.float32),
        input_output_aliases={0: 0},
    )(out, e, W, b2)
    return out


def kernel(x, table, W, b):
    idx = x.astype(jnp.int32)
    e = _gather_sc(table, idx)
    return _project(e, W, b)


# speedup vs baseline: 1.0073x; 1.0073x over previous
import functools

import jax
import jax.numpy as jnp
from jax import lax
from jax.experimental import pallas as pl
from jax.experimental.pallas import tpu as pltpu
from jax.experimental.pallas import tpu_sc as plsc

VOCAB = 100000
DIM = 128
BATCH = 1024


def _gather_sc(table, idx):
    info = plsc.get_sparse_core_info()
    nw = info.num_cores * info.num_subcores
    bpw = BATCH // nw
    mesh = plsc.VectorSubcoreMesh(core_axis_name="c", subcore_axis_name="s")

    @functools.partial(
        pl.kernel,
        mesh=mesh,
        out_type=jax.ShapeDtypeStruct((BATCH, DIM), jnp.float32),
        scratch_types=[
            pltpu.VMEM((bpw,), jnp.int32),
            pltpu.VMEM((bpw, DIM), jnp.float32),
            pltpu.SemaphoreType.DMA,
        ],
    )
    def gather_kernel(table_hbm, idx_hbm, out_hbm, idx_v, rows_v, sem):
        wid = lax.axis_index("s") * info.num_cores + lax.axis_index("c")
        base = wid * bpw
        pltpu.sync_copy(idx_hbm.at[pl.ds(base, bpw)], idx_v)
        pltpu.async_copy(table_hbm.at[idx_v], rows_v, sem).wait()
        pltpu.sync_copy(rows_v, out_hbm.at[pl.ds(base, bpw)])

    return gather_kernel(table, idx)


_VT = 2048
_NT = (VOCAB + _VT - 1) // _VT
_NFULL = VOCAB // _VT
_NB = 4


_TA = 1664
_TB = 128


def _proj_kernel(e_ref, w_ref, b_ref, o_hbm, *scratch):
    bufs = scratch[:_NB]
    sems = scratch[_NB : 2 * _NB]
    buf_t, sem_a, sem_b = scratch[2 * _NB :]
    j = pl.program_id(0)
    acc = (
        jnp.dot(e_ref[...], w_ref[...], preferred_element_type=jnp.float32)
        + b_ref[...]
    )

    @pl.when(j < _NFULL)
    def _():
        for s in range(_NB):

            @pl.when(jax.lax.rem(j, _NB) == s)
            def _(s=s):
                @pl.when(j >= _NB)
                def _():
                    pltpu.make_async_copy(
                        bufs[s], o_hbm.at[:, pl.ds((j - _NB) * _VT, _VT)], sems[s]
                    ).wait()

                bufs[s][...] = acc
                pltpu.make_async_copy(
                    bufs[s], o_hbm.at[:, pl.ds(j * _VT, _VT)], sems[s]
                ).start()

    @pl.when(j == _NT - 1)
    def _():
        buf_t[...] = acc
        pltpu.make_async_copy(
            buf_t.at[:, pl.ds(0, _TA)], o_hbm.at[:, pl.ds(j * _VT, _TA)], sem_a
        ).start()
        start = pl.multiple_of(j * _VT + _TA, 128)
        pltpu.make_async_copy(
            buf_t.at[:, pl.ds(_TA, _TB)], o_hbm.at[:, pl.ds(start, _TB)], sem_b
        ).start()
        for s in range(_NB):
            jl = _NFULL - 1 - ((_NFULL - 1 - s) % _NB)
            pltpu.make_async_copy(
                bufs[s], o_hbm.at[:, pl.ds(jl * _VT, _VT)], sems[s]
            ).wait()
        pltpu.make_async_copy(
            buf_t.at[:, pl.ds(0, _TA)], o_hbm.at[:, pl.ds(j * _VT, _TA)], sem_a
        ).wait()
        pltpu.make_async_copy(
            buf_t.at[:, pl.ds(_TA, _TB)], o_hbm.at[:, pl.ds(start, _TB)], sem_b
        ).wait()


def _project(e, W, b):
    b2 = b.reshape(1, VOCAB)
    return pl.pallas_call(
        _proj_kernel,
        grid=(_NT,),
        in_specs=[
            pl.BlockSpec((BATCH, DIM), lambda j: (0, 0)),
            pl.BlockSpec((DIM, _VT), lambda j: (0, j)),
            pl.BlockSpec((1, _VT), lambda j: (0, j)),
        ],
        out_specs=pl.BlockSpec(memory_space=pl.ANY),
        out_shape=jax.ShapeDtypeStruct((BATCH, VOCAB), jnp.float32),
        scratch_shapes=(
            [pltpu.VMEM((BATCH, _VT), jnp.float32) for _ in range(_NB)]
            + [pltpu.SemaphoreType.DMA for _ in range(_NB)]
            + [
                pltpu.VMEM((BATCH, _VT), jnp.float32),
                pltpu.SemaphoreType.DMA,
                pltpu.SemaphoreType.DMA,
            ]
        ),
    )(e, W, b2)


def kernel(x, table, W, b):
    idx = x.astype(jnp.int32)
    e = _gather_sc(table, idx)
    return _project(e, W, b)
